# baseline (speedup 1.0000x reference)
import functools

import jax
import jax.numpy as jnp
from jax import lax
from jax.experimental import pallas as pl
from jax.experimental.pallas import tpu as pltpu
from jax.experimental.pallas import tpu_sc as plsc

OLD = 100000
D = 128
NC = 2
NS = 16
NW = NC * NS
LANES = 16

TOTAL = 16384
RPT = TOTAL // NW
GCH = 128
BATCH = 64


def _mlp_body(nv_ref, nu_ref, ut_ref, w1_ref, b1_ref, w2_ref, b2_ref, o_ref):
  nv = nv_ref[...]
  nu = nu_ref[...]
  blk = nv.shape[0]
  iota = lax.broadcasted_iota(jnp.int32, (blk, 8), 1)
  onehot = (iota == nu).astype(jnp.float32)
  ut2 = ut_ref[:, 0:2]
  w1 = w1_ref[...]
  m = jnp.dot(ut2, w1[1:3, :], preferred_element_type=jnp.float32)
  h_pre = nv * w1[0:1, :] + jnp.dot(onehot, m, preferred_element_type=jnp.float32) + b1_ref[...]
  h = 0.5 * h_pre * (1.0 + lax.erf(h_pre * 0.7071067811865476))
  o_ref[...] = jnp.dot(h, w2_ref[...], preferred_element_type=jnp.float32) + b2_ref[...]


def _mlp_rows(num_values, num_units, unit_table, W1, b1, W2, b2, n_pad):
  blk = min(n_pad, 512)
  grid = n_pad // blk
  nv = jnp.zeros((n_pad, 1), jnp.float32).at[: num_values.shape[0], 0].set(num_values)
  nu = jnp.full((n_pad, 1), 0, jnp.int32).at[: num_units.shape[0], 0].set(num_units)
  ut_pad = jnp.zeros((8, 128), jnp.float32).at[:6, :2].set(unit_table)
  w1_pad = jnp.zeros((8, W1.shape[1]), jnp.float32).at[:3, :].set(W1)
  return pl.pallas_call(
      _mlp_body,
      grid=(grid,),
      in_specs=[
          pl.BlockSpec((blk, 1), lambda i: (i, 0)),
          pl.BlockSpec((blk, 1), lambda i: (i, 0)),
          pl.BlockSpec((8, 128), lambda i: (0, 0)),
          pl.BlockSpec((8, 512), lambda i: (0, 0)),
          pl.BlockSpec((1, 512), lambda i: (0, 0)),
          pl.BlockSpec((512, 128), lambda i: (0, 0)),
          pl.BlockSpec((1, 128), lambda i: (0, 0)),
      ],
      out_specs=pl.BlockSpec((blk, 128), lambda i: (i, 0)),
      out_shape=jax.ShapeDtypeStruct((n_pad, 128), jnp.float32),
  )(nv, nu, ut_pad, w1_pad, b1.reshape(1, 512), W2, b2.reshape(1, 128))


def _sc_kernel_factory(ls, ln):
  mesh = plsc.VectorSubcoreMesh(core_axis_name="c", subcore_axis_name="s")

  @functools.partial(
      pl.kernel,
      out_type=jax.ShapeDtypeStruct((TOTAL, D), jnp.float32),
      mesh=mesh,
      compiler_params=pltpu.CompilerParams(needs_layout_passes=False),
      scratch_types=[
          pltpu.VMEM((RPT,), jnp.int32),
          pltpu.VMEM((RPT, D), jnp.float32),
          pltpu.VMEM((ls,), jnp.int32),
          pltpu.VMEM((ln,), jnp.int32),
          pltpu.VMEM((RPT + 16,), jnp.int32),
          pltpu.VMEM((RPT + 16,), jnp.int32),
          pltpu.VMEM((BATCH, D), jnp.float32),
          pltpu.VMEM((BATCH + 8, D), jnp.float32),
          pltpu.SemaphoreType.DMA,
      ],
  )
  def sc_kernel(ids_hbm, sp_hbm, np_hbm, mlp_hbm, orig_hbm, new_hbm, out_hbm,
                ids_v, rows_v, sp_v, np_v, sel_buf, off_buf, nrows_v, mrows_v,
                sem):
    wid = lax.axis_index("s") * NC + lax.axis_index("c")
    base = wid * RPT

    zv = jnp.zeros((LANES,), jnp.int32)
    for z in range((RPT + 16) // LANES):
      sel_buf[pl.ds(z * LANES, LANES)] = zv

    pltpu.sync_copy(ids_hbm.at[pl.ds(base, RPT)], ids_v)
    pltpu.sync_copy(sp_hbm, sp_v)
    pltpu.sync_copy(np_hbm, np_v)

    cps = []
    for k in range(RPT // GCH):
      cps.append(pltpu.async_copy(
          orig_hbm.at[ids_v.at[pl.ds(k * GCH, GCH)]],
          rows_v.at[pl.ds(k * GCH, GCH)], sem))
    for cp in cps:
      cp.wait()

    def scan_list(list_v, nchunks):
      def body(j, carry):
        cnt, r0 = carry
        pos = list_v[pl.ds(j * LANES, LANES)]
        m = (pos >= base) & (pos < base + RPT)
        off = jnp.where(m, pos - base, 0)
        idv = plsc.load_gather(ids_v, [off], mask=m)
        sel = jnp.where(m, idv - OLD, 0)
        plsc.store_compressed(sel_buf.at[pl.ds(cnt, LANES)], sel, mask=m)
        plsc.store_compressed(off_buf.at[pl.ds(cnt, LANES)], off, mask=m)
        cnt = cnt + jnp.sum(m.astype(jnp.int32))
        r0 = r0 + jnp.sum((pos < base).astype(jnp.int32))
        return cnt, r0
      return lax.fori_loop(0, nchunks, body, (jnp.int32(0), jnp.int32(0)))

    def patch_rows(cnt, r0, with_mlp):
      r0a = pl.multiple_of((r0 // 8) * 8, 8)
      shift = r0 - r0a

      def batch_body(b, _):
        ib = b * BATCH
        pltpu.async_copy(new_hbm.at[sel_buf.at[pl.ds(ib, BATCH)]],
                         nrows_v, sem).wait()
        if with_mlp:
          pltpu.sync_copy(mlp_hbm.at[pl.ds(r0a + ib, BATCH + 8)], mrows_v)
        nin = jnp.minimum(BATCH, cnt - ib)

        def row_body(e, _):
          off = off_buf[pl.ds(ib + e, LANES)][0]
          for k in range(D // LANES):
            v = nrows_v[e, pl.ds(k * LANES, LANES)]
            if with_mlp:
              v = v + mrows_v[e + shift, pl.ds(k * LANES, LANES)]
            rows_v[off, pl.ds(k * LANES, LANES)] = v
          return 0
        lax.fori_loop(0, nin, row_body, 0)
        return 0
      lax.fori_loop(0, (cnt + BATCH - 1) // BATCH, batch_body, 0)

    cnt_s, _ = scan_list(sp_v, ls // LANES)
    patch_rows(cnt_s, jnp.int32(0), False)
    cnt_n, r0_n = scan_list(np_v, ln // LANES)
    patch_rows(cnt_n, r0_n, True)

    pltpu.sync_copy(rows_v, out_hbm.at[pl.ds(base, RPT)])

  return sc_kernel


def _ceil16(n):
  return max(16, (n + 15) // 16 * 16)


def kernel(input_ids, stocks_pos, num_pos, num_values, num_units,
           orig_table, new_table, unit_table, W1, b1, W2, b2):
  ids_flat = input_ids.reshape(-1)
  n_s = stocks_pos.shape[0]
  n_n = num_pos.shape[0]
  ls, ln = _ceil16(n_s), _ceil16(n_n)
  big = jnp.int32(1 << 30)
  sp_pad = jnp.full((ls,), big, jnp.int32).at[:n_s].set(stocks_pos)
  np_pad = jnp.full((ln,), big, jnp.int32).at[:n_n].set(num_pos)

  n_pad = max(512, (n_n + 511) // 512 * 512) + 512
  mlp = _mlp_rows(num_values, num_units, unit_table, W1, b1, W2, b2, n_pad)

  sc = _sc_kernel_factory(ls, ln)
  out = sc(ids_flat, sp_pad, np_pad, mlp, orig_table, new_table)
  return out.reshape(input_ids.shape[0], input_ids.shape[1], D)


# speedup vs baseline: 1.0002x; 1.0002x over previous
import functools

import jax
import jax.numpy as jnp
from jax import lax
from jax.experimental import pallas as pl
from jax.experimental.pallas import tpu as pltpu
from jax.experimental.pallas import tpu_sc as plsc

OLD = 100000
D = 128
NC = 2
NS = 16
NW = NC * NS
LANES = 16

TOTAL = 16384
RPT = TOTAL // NW
GCH = 128
BATCH = 64


def _mlp_body(nv_ref, nu_ref, ut_ref, w1_ref, b1_ref, w2_ref, b2_ref, o_ref):
  nv = nv_ref[...]
  nu = nu_ref[...]
  blk = nv.shape[0]
  iota = lax.broadcasted_iota(jnp.int32, (blk, 8), 1)
  onehot = (iota == nu).astype(jnp.float32)
  ut2 = ut_ref[:, 0:2]
  w1 = w1_ref[...]
  m = jnp.dot(ut2, w1[1:3, :], preferred_element_type=jnp.float32)
  h_pre = nv * w1[0:1, :] + jnp.dot(onehot, m, preferred_element_type=jnp.float32) + b1_ref[...]
  h = 0.5 * h_pre * (1.0 + lax.erf(h_pre * 0.7071067811865476))
  o_ref[...] = jnp.dot(h, w2_ref[...], preferred_element_type=jnp.float32) + b2_ref[...]


def _mlp_rows(num_values, num_units, unit_table, W1, b1, W2, b2, n_pad):
  blk = min(n_pad, 512)
  grid = n_pad // blk
  nv = jnp.zeros((n_pad, 1), jnp.float32).at[: num_values.shape[0], 0].set(num_values)
  nu = jnp.full((n_pad, 1), 0, jnp.int32).at[: num_units.shape[0], 0].set(num_units)
  ut_pad = jnp.zeros((8, 128), jnp.float32).at[:6, :2].set(unit_table)
  w1_pad = jnp.zeros((8, W1.shape[1]), jnp.float32).at[:3, :].set(W1)
  return pl.pallas_call(
      _mlp_body,
      grid=(grid,),
      in_specs=[
          pl.BlockSpec((blk, 1), lambda i: (i, 0)),
          pl.BlockSpec((blk, 1), lambda i: (i, 0)),
          pl.BlockSpec((8, 128), lambda i: (0, 0)),
          pl.BlockSpec((8, 512), lambda i: (0, 0)),
          pl.BlockSpec((1, 512), lambda i: (0, 0)),
          pl.BlockSpec((512, 128), lambda i: (0, 0)),
          pl.BlockSpec((1, 128), lambda i: (0, 0)),
      ],
      out_specs=pl.BlockSpec((blk, 128), lambda i: (i, 0)),
      out_shape=jax.ShapeDtypeStruct((n_pad, 128), jnp.float32),
  )(nv, nu, ut_pad, w1_pad, b1.reshape(1, 512), W2, b2.reshape(1, 128))


def _sc_kernel_factory(ls, ln):
  mesh = plsc.VectorSubcoreMesh(core_axis_name="c", subcore_axis_name="s")

  @functools.partial(
      pl.kernel,
      out_type=jax.ShapeDtypeStruct((TOTAL, D), jnp.float32),
      mesh=mesh,
      compiler_params=pltpu.CompilerParams(needs_layout_passes=False),
      scratch_types=[
          pltpu.VMEM((RPT,), jnp.int32),
          pltpu.VMEM((RPT, D), jnp.float32),
          pltpu.VMEM((ls,), jnp.int32),
          pltpu.VMEM((ln,), jnp.int32),
          pltpu.VMEM((RPT + 16,), jnp.int32),
          pltpu.VMEM((RPT + 16,), jnp.int32),
          pltpu.VMEM((BATCH, D), jnp.float32),
          pltpu.VMEM((BATCH + 8, D), jnp.float32),
          pltpu.SemaphoreType.DMA,
      ],
  )
  def sc_kernel(ids_hbm, sp_hbm, np_hbm, mlp_hbm, orig_hbm, new_hbm, out_hbm,
                ids_v, rows_v, sp_v, np_v, sel_buf, off_buf, nrows_v, mrows_v,
                sem):
    wid = lax.axis_index("s") * NC + lax.axis_index("c")
    base = wid * RPT

    with jax.named_scope("zinit"):
      zv = jnp.zeros((LANES,), jnp.int32)
      for z in range((RPT + 16) // LANES):
        sel_buf[pl.ds(z * LANES, LANES)] = zv

    with jax.named_scope("stage"):
      pltpu.sync_copy(ids_hbm.at[pl.ds(base, RPT)], ids_v)
      pltpu.sync_copy(sp_hbm, sp_v)
      pltpu.sync_copy(np_hbm, np_v)

    with jax.named_scope("gather"):
      cps = []
      for k in range(RPT // GCH):
        cps.append(pltpu.async_copy(
            orig_hbm.at[ids_v.at[pl.ds(k * GCH, GCH)]],
            rows_v.at[pl.ds(k * GCH, GCH)], sem))
      for cp in cps:
        cp.wait()

    def scan_list(list_v, nchunks):
      def body(j, carry):
        cnt, r0 = carry
        pos = list_v[pl.ds(j * LANES, LANES)]
        m = (pos >= base) & (pos < base + RPT)
        off = jnp.where(m, pos - base, 0)
        idv = plsc.load_gather(ids_v, [off], mask=m)
        sel = jnp.where(m, idv - OLD, 0)
        plsc.store_compressed(sel_buf.at[pl.ds(cnt, LANES)], sel, mask=m)
        plsc.store_compressed(off_buf.at[pl.ds(cnt, LANES)], off, mask=m)
        cnt = cnt + jnp.sum(m.astype(jnp.int32))
        r0 = r0 + jnp.sum((pos < base).astype(jnp.int32))
        return cnt, r0
      return lax.fori_loop(0, nchunks, body, (jnp.int32(0), jnp.int32(0)))

    def patch_rows(cnt, r0, with_mlp):
      r0a = pl.multiple_of((r0 // 8) * 8, 8)
      shift = r0 - r0a

      def batch_body(b, _):
        ib = b * BATCH
        pltpu.async_copy(new_hbm.at[sel_buf.at[pl.ds(ib, BATCH)]],
                         nrows_v, sem).wait()
        if with_mlp:
          pltpu.sync_copy(mlp_hbm.at[pl.ds(r0a + ib, BATCH + 8)], mrows_v)
        nin = jnp.minimum(BATCH, cnt - ib)

        def row_body(e, _):
          off = off_buf[pl.ds(ib + e, LANES)][0]
          for k in range(D // LANES):
            v = nrows_v[e, pl.ds(k * LANES, LANES)]
            if with_mlp:
              v = v + mrows_v[e + shift, pl.ds(k * LANES, LANES)]
            rows_v[off, pl.ds(k * LANES, LANES)] = v
          return 0
        lax.fori_loop(0, nin, row_body, 0)
        return 0
      lax.fori_loop(0, (cnt + BATCH - 1) // BATCH, batch_body, 0)

    with jax.named_scope("scan_s"):
      cnt_s, _ = scan_list(sp_v, ls // LANES)
    with jax.named_scope("patch_s"):
      patch_rows(cnt_s, jnp.int32(0), False)
    with jax.named_scope("scan_n"):
      cnt_n, r0_n = scan_list(np_v, ln // LANES)
    with jax.named_scope("patch_n"):
      patch_rows(cnt_n, r0_n, True)

    with jax.named_scope("writeout"):
      pltpu.sync_copy(rows_v, out_hbm.at[pl.ds(base, RPT)])

  return sc_kernel


def _ceil16(n):
  return max(16, (n + 15) // 16 * 16)


def kernel(input_ids, stocks_pos, num_pos, num_values, num_units,
           orig_table, new_table, unit_table, W1, b1, W2, b2):
  ids_flat = input_ids.reshape(-1)
  n_s = stocks_pos.shape[0]
  n_n = num_pos.shape[0]
  ls, ln = _ceil16(n_s), _ceil16(n_n)
  big = jnp.int32(1 << 30)
  sp_pad = jnp.full((ls,), big, jnp.int32).at[:n_s].set(stocks_pos)
  np_pad = jnp.full((ln,), big, jnp.int32).at[:n_n].set(num_pos)

  n_pad = max(512, (n_n + 511) // 512 * 512) + 512
  mlp = _mlp_rows(num_values, num_units, unit_table, W1, b1, W2, b2, n_pad)

  sc = _sc_kernel_factory(ls, ln)
  out = sc(ids_flat, sp_pad, np_pad, mlp, orig_table, new_table)
  return out.reshape(input_ids.shape[0], input_ids.shape[1], D)


# speedup vs baseline: 1.4429x; 1.4426x over previous
import functools

import jax
import jax.numpy as jnp
from jax import lax
from jax.experimental import pallas as pl
from jax.experimental.pallas import tpu as pltpu
from jax.experimental.pallas import tpu_sc as plsc

OLD = 100000
D = 128
NC = 2
NS = 16
NW = NC * NS
LANES = 16

TOTAL = 16384
RPT = TOTAL // NW
GCH = 128
NGC = RPT // GCH
EB = 64
DUMMY = TOTAL


def _mlp_body(nv_ref, nu_ref, ut_ref, w1_ref, b1_ref, w2_ref, b2_ref, o_ref):
  nv = nv_ref[...]
  nu = nu_ref[...]
  blk = nv.shape[0]
  iota = lax.broadcasted_iota(jnp.int32, (blk, 8), 1)
  onehot = (iota == nu).astype(jnp.float32)
  ut2 = ut_ref[:, 0:2]
  w1 = w1_ref[...]
  m = jnp.dot(ut2, w1[1:3, :], preferred_element_type=jnp.float32)
  h_pre = nv * w1[0:1, :] + jnp.dot(onehot, m, preferred_element_type=jnp.float32) + b1_ref[...]
  h = 0.5 * h_pre * (1.0 + lax.erf(h_pre * 0.7071067811865476))
  o_ref[...] = jnp.dot(h, w2_ref[...], preferred_element_type=jnp.float32) + b2_ref[...]


def _mlp_rows(num_values, num_units, unit_table, W1, b1, W2, b2, n_pad):
  blk = min(n_pad, 512)
  grid = n_pad // blk
  nv = jnp.zeros((n_pad, 1), jnp.float32).at[: num_values.shape[0], 0].set(num_values)
  nu = jnp.full((n_pad, 1), 0, jnp.int32).at[: num_units.shape[0], 0].set(num_units)
  ut_pad = jnp.zeros((8, 128), jnp.float32).at[:6, :2].set(unit_table)
  w1_pad = jnp.zeros((8, W1.shape[1]), jnp.float32).at[:3, :].set(W1)
  return pl.pallas_call(
      _mlp_body,
      grid=(grid,),
      in_specs=[
          pl.BlockSpec((blk, 1), lambda i: (i, 0)),
          pl.BlockSpec((blk, 1), lambda i: (i, 0)),
          pl.BlockSpec((8, 128), lambda i: (0, 0)),
          pl.BlockSpec((8, 512), lambda i: (0, 0)),
          pl.BlockSpec((1, 512), lambda i: (0, 0)),
          pl.BlockSpec((512, 128), lambda i: (0, 0)),
          pl.BlockSpec((1, 128), lambda i: (0, 0)),
      ],
      out_specs=pl.BlockSpec((blk, 128), lambda i: (i, 0)),
      out_shape=jax.ShapeDtypeStruct((n_pad, 128), jnp.float32),
  )(nv, nu, ut_pad, w1_pad, b1.reshape(1, 512), W2, b2.reshape(1, 128))


def _sc_kernel_factory(nbs, nbn):
  mesh = plsc.VectorSubcoreMesh(core_axis_name="c", subcore_axis_name="s")

  @functools.partial(
      pl.kernel,
      out_type=jax.ShapeDtypeStruct((TOTAL + 16, D), jnp.float32),
      mesh=mesh,
      compiler_params=pltpu.CompilerParams(needs_layout_passes=False),
      scratch_types=[
          pltpu.VMEM((RPT,), jnp.int32),
          pltpu.VMEM((NGC, GCH), jnp.int32),
          pltpu.VMEM((RPT, D), jnp.float32),
          pltpu.VMEM((1, EB), jnp.int32),
          pltpu.VMEM((EB,), jnp.int32),
          pltpu.VMEM((EB,), jnp.int32),
          pltpu.VMEM((EB, D), jnp.float32),
          pltpu.VMEM((EB, D), jnp.float32),
          [pltpu.SemaphoreType.DMA] * 4,
          pltpu.SemaphoreType.DMA,
          pltpu.SemaphoreType.DMA,
          pltpu.SemaphoreType.DMA,
      ],
  )
  def sc_kernel(ids_hbm, sp_hbm, np_hbm, mlp_hbm, orig_hbm, new_hbm, out_hbm,
                ids_v, tgt_v, rows_v, pos2d, idsel_v, sel_v, nrows_v, mrows_v,
                gsems, ssem, psem, ssem2):
    t = lax.axis_index("s") * NC + lax.axis_index("c")
    base = t * RPT

    with jax.named_scope("stage_ids"):
      pltpu.sync_copy(ids_hbm.at[pl.ds(base, RPT)], ids_v)

    with jax.named_scope("build_tgt"):
      for j in range(NGC):
        for c in range(GCH // LANES):
          i0 = j * GCH + c * LANES
          idv = ids_v[pl.ds(i0, LANES)]
          pos = base + i0 + lax.iota(jnp.int32, LANES)
          tgt_v[j, pl.ds(c * LANES, LANES)] = jnp.where(idv < OLD, pos, DUMMY)

    with jax.named_scope("gather_scatter"):
      cps = []
      for j in range(NGC):
        cps.append(pltpu.async_copy(
            orig_hbm.at[ids_v.at[pl.ds(j * GCH, GCH)]],
            rows_v.at[pl.ds(j * GCH, GCH)], gsems[j]))
      for j in range(NGC):
        cps[j].wait()
        pltpu.async_copy(rows_v.at[pl.ds(j * GCH, GCH)],
                         out_hbm.at[tgt_v.at[j]], ssem)

    with jax.named_scope("overwrite"):
      def make_batch(list_hbm, nb, with_mlp):
        def batch_body(i, _):
          g = t + i * NW
          pltpu.sync_copy(list_hbm.at[pl.ds(g * EB, EB)], pos2d.at[0])
          pltpu.async_copy(ids_hbm.at[pos2d.at[0]], idsel_v, psem).wait()
          for c in range(EB // LANES):
            sel_v[pl.ds(c * LANES, LANES)] = (
                idsel_v[pl.ds(c * LANES, LANES)] - OLD)
          cpn = pltpu.async_copy(new_hbm.at[sel_v], nrows_v, psem)
          if with_mlp:
            pltpu.sync_copy(mlp_hbm.at[pl.ds(g * EB, EB)], mrows_v)
          cpn.wait()
          if with_mlp:
            def add_row(e, _):
              for k in range(D // LANES):
                nrows_v[e, pl.ds(k * LANES, LANES)] = (
                    nrows_v[e, pl.ds(k * LANES, LANES)]
                    + mrows_v[e, pl.ds(k * LANES, LANES)])
              return 0
            lax.fori_loop(0, EB, add_row, 0)
          pltpu.async_copy(nrows_v, out_hbm.at[pos2d.at[0]], ssem2).wait()
          return 0
        nmine = jnp.maximum(0, (nb - t + NW - 1) // NW)
        lax.fori_loop(0, nmine, batch_body, 0)

      make_batch(sp_hbm, nbs, False)
      make_batch(np_hbm, nbn, True)

    with jax.named_scope("drain"):
      for j in range(NGC):
        pltpu.make_async_copy(rows_v.at[pl.ds(j * GCH, GCH)],
                              out_hbm.at[tgt_v.at[j]], ssem).wait()

  return sc_kernel


def _ceil(n, m):
  return max(m, (n + m - 1) // m * m)


def kernel(input_ids, stocks_pos, num_pos, num_values, num_units,
           orig_table, new_table, unit_table, W1, b1, W2, b2):
  ids_flat = input_ids.reshape(-1)
  ids_pad = jnp.concatenate(
      [ids_flat, jnp.full((16,), OLD, jnp.int32)])
  n_s = stocks_pos.shape[0]
  n_n = num_pos.shape[0]
  ls, ln = _ceil(n_s, EB), _ceil(n_n, EB)
  sp_pad = jnp.full((ls,), DUMMY, jnp.int32).at[:n_s].set(stocks_pos)
  np_pad = jnp.full((ln,), DUMMY, jnp.int32).at[:n_n].set(num_pos)

  n_pad = _ceil(ln, 512)
  mlp = _mlp_rows(num_values, num_units, unit_table, W1, b1, W2, b2, n_pad)

  sc = _sc_kernel_factory(ls // EB, ln // EB)
  out = sc(ids_pad, sp_pad, np_pad, mlp, orig_table, new_table)
  return out[:TOTAL].reshape(input_ids.shape[0], input_ids.shape[1], D)


# speedup vs baseline: 1.4653x; 1.0155x over previous
import functools

import jax
import jax.numpy as jnp
from jax import lax
from jax.experimental import pallas as pl
from jax.experimental.pallas import tpu as pltpu
from jax.experimental.pallas import tpu_sc as plsc

OLD = 100000
D = 128
NC = 2
NS = 16
NW = NC * NS
LANES = 16

TOTAL = 16384
RPT = TOTAL // NW
GCH = 128
NGC = RPT // GCH
EB = 64
DUMMY = TOTAL


def _mlp_body(nv_ref, nu_ref, ut_ref, w1_ref, b1_ref, w2_ref, b2_ref, o_ref):
  nv = nv_ref[...]
  nu = nu_ref[...]
  blk = nv.shape[0]
  iota = lax.broadcasted_iota(jnp.int32, (blk, 8), 1)
  onehot = (iota == nu).astype(jnp.float32)
  ut2 = ut_ref[:, 0:2]
  w1 = w1_ref[...]
  m = jnp.dot(ut2, w1[1:3, :], preferred_element_type=jnp.float32)
  h_pre = nv * w1[0:1, :] + jnp.dot(onehot, m, preferred_element_type=jnp.float32) + b1_ref[...]
  h = 0.5 * h_pre * (1.0 + lax.erf(h_pre * 0.7071067811865476))
  o_ref[...] = jnp.dot(h, w2_ref[...], preferred_element_type=jnp.float32) + b2_ref[...]


def _mlp_rows(num_values, num_units, unit_table, W1, b1, W2, b2, n_pad):
  blk = min(n_pad, 512)
  grid = n_pad // blk
  nv = jnp.zeros((n_pad, 1), jnp.float32).at[: num_values.shape[0], 0].set(num_values)
  nu = jnp.full((n_pad, 1), 0, jnp.int32).at[: num_units.shape[0], 0].set(num_units)
  ut_pad = jnp.zeros((8, 128), jnp.float32).at[:6, :2].set(unit_table)
  w1_pad = jnp.zeros((8, W1.shape[1]), jnp.float32).at[:3, :].set(W1)
  return pl.pallas_call(
      _mlp_body,
      grid=(grid,),
      in_specs=[
          pl.BlockSpec((blk, 1), lambda i: (i, 0)),
          pl.BlockSpec((blk, 1), lambda i: (i, 0)),
          pl.BlockSpec((8, 128), lambda i: (0, 0)),
          pl.BlockSpec((8, 512), lambda i: (0, 0)),
          pl.BlockSpec((1, 512), lambda i: (0, 0)),
          pl.BlockSpec((512, 128), lambda i: (0, 0)),
          pl.BlockSpec((1, 128), lambda i: (0, 0)),
      ],
      out_specs=pl.BlockSpec((blk, 128), lambda i: (i, 0)),
      out_shape=jax.ShapeDtypeStruct((n_pad, 128), jnp.float32),
  )(nv, nu, ut_pad, w1_pad, b1.reshape(1, 512), W2, b2.reshape(1, 128))


def _sc_kernel_factory(nbs, nbn):
  mesh = plsc.VectorSubcoreMesh(core_axis_name="c", subcore_axis_name="s")

  @functools.partial(
      pl.kernel,
      out_type=jax.ShapeDtypeStruct((TOTAL + 16, D), jnp.float32),
      mesh=mesh,
      compiler_params=pltpu.CompilerParams(needs_layout_passes=False),
      scratch_types=[
          pltpu.VMEM((RPT,), jnp.int32),
          pltpu.VMEM((NGC, GCH), jnp.int32),
          pltpu.VMEM((RPT, D), jnp.float32),
          pltpu.VMEM((1, EB), jnp.int32),
          pltpu.VMEM((EB,), jnp.int32),
          pltpu.VMEM((EB,), jnp.int32),
          pltpu.VMEM((EB,), jnp.int32),
          pltpu.VMEM((EB, D), jnp.float32),
          pltpu.VMEM((EB, D), jnp.float32),
          [pltpu.SemaphoreType.DMA] * 4,
          pltpu.SemaphoreType.DMA,
          pltpu.SemaphoreType.DMA,
          pltpu.SemaphoreType.DMA,
      ],
  )
  def sc_kernel(ids_hbm, list_hbm, mlp_hbm, orig_hbm, new_hbm, out_hbm,
                ids_v, tgt_v, rows_v, pos2d, posg_v, idsel_v, sel_v,
                nrows_v, mrows_v, gsems, ssem, psem, ssem2):
    t = lax.axis_index("s") * NC + lax.axis_index("c")
    base = t * RPT

    with jax.named_scope("stage_ids"):
      pltpu.sync_copy(ids_hbm.at[pl.ds(base, RPT)], ids_v)

    with jax.named_scope("gather_fire"):
      cps = []
      for j in range(NGC):
        cps.append(pltpu.async_copy(
            orig_hbm.at[ids_v.at[pl.ds(j * GCH, GCH)]],
            rows_v.at[pl.ds(j * GCH, GCH)], gsems[j]))

    with jax.named_scope("build_tgt"):
      for j in range(NGC):
        for c in range(GCH // LANES):
          i0 = j * GCH + c * LANES
          idv = ids_v[pl.ds(i0, LANES)]
          pos = base + i0 + lax.iota(jnp.int32, LANES)
          tgt_v[j, pl.ds(c * LANES, LANES)] = jnp.where(idv < OLD, pos, DUMMY)

    with jax.named_scope("overwrite"):
      def batch_body(i, _):
        g = t + i * NW
        pltpu.sync_copy(list_hbm.at[pl.ds(g * EB, EB)], pos2d.at[0])
        for c in range(EB // LANES):
          posg_v[pl.ds(c * LANES, LANES)] = jnp.minimum(
              pos2d[0, pl.ds(c * LANES, LANES)], TOTAL - 1)
        pltpu.async_copy(ids_hbm.at[posg_v], idsel_v, psem).wait()
        for c in range(EB // LANES):
          sel_v[pl.ds(c * LANES, LANES)] = jnp.clip(
              idsel_v[pl.ds(c * LANES, LANES)] - OLD, 0, 9999)
        cpn = pltpu.async_copy(new_hbm.at[sel_v], nrows_v, psem)
        is_num = g >= nbs

        @pl.when(is_num)
        def _():
          pltpu.sync_copy(mlp_hbm.at[pl.ds((g - nbs) * EB, EB)], mrows_v)
        cpn.wait()

        @pl.when(is_num)
        def _():
          def add_row(e, _):
            for k in range(D // LANES):
              nrows_v[e, pl.ds(k * LANES, LANES)] = (
                  nrows_v[e, pl.ds(k * LANES, LANES)]
                  + mrows_v[e, pl.ds(k * LANES, LANES)])
            return 0
          lax.fori_loop(0, EB, add_row, 0)
        pltpu.async_copy(nrows_v, out_hbm.at[pos2d.at[0]], ssem2).wait()
        return 0

      nbt = nbs + nbn
      nmine = jnp.maximum(0, (nbt - t + NW - 1) // NW)
      lax.fori_loop(0, nmine, batch_body, 0)

    with jax.named_scope("gather_scatter"):
      for j in range(NGC):
        cps[j].wait()
        pltpu.async_copy(rows_v.at[pl.ds(j * GCH, GCH)],
                         out_hbm.at[tgt_v.at[j]], ssem)

    with jax.named_scope("drain"):
      for j in range(NGC):
        pltpu.make_async_copy(rows_v.at[pl.ds(j * GCH, GCH)],
                              out_hbm.at[tgt_v.at[j]], ssem).wait()

  return sc_kernel


def _ceil(n, m):
  return max(m, (n + m - 1) // m * m)


def kernel(input_ids, stocks_pos, num_pos, num_values, num_units,
           orig_table, new_table, unit_table, W1, b1, W2, b2):
  ids_flat = input_ids.reshape(-1)
  n_s = stocks_pos.shape[0]
  n_n = num_pos.shape[0]
  ls, ln = _ceil(n_s, EB), _ceil(n_n, EB)
  lists = jnp.full((ls + ln,), DUMMY, jnp.int32)
  lists = lists.at[:n_s].set(stocks_pos).at[ls:ls + n_n].set(num_pos)

  n_pad = _ceil(ln, 512)
  mlp = _mlp_rows(num_values, num_units, unit_table, W1, b1, W2, b2, n_pad)

  sc = _sc_kernel_factory(ls // EB, ln // EB)
  out = sc(ids_flat, lists, mlp, orig_table, new_table)
  return out[:TOTAL].reshape(input_ids.shape[0], input_ids.shape[1], D)
